# Initial kernel scaffold; baseline (speedup 1.0000x reference)
import jax
import jax.numpy as jnp
from jax.experimental import pallas as pl


def kernel(x, e, fc_W, fc_b, W0, b0, W1, b1, Wr, Ur, br, Wu, Uu, bu, Wc, Uc, bc):
    raise NotImplementedError("write your pallas kernel here")


# speedup vs baseline: 1.1902x; 1.1902x over previous
import jax
import jax.numpy as jnp
from jax.experimental import pallas as pl
from jax.experimental.pallas import tpu as pltpu


def _step(x_ref, e_ref, fcW_ref, fcb_ref, W0_ref, b0_ref, W1_ref, b1_ref,
          Wr_ref, Ur_ref, br_ref, Wu_ref, Uu_ref, bu_ref,
          Wc_ref, Uc_ref, bc_ref, out_ref, h_s):
    t = pl.program_id(0)
    b = pl.program_id(1)

    A = e_ref[0, 0]
    x = x_ref[0, 0]

    dot = lambda a, w: jnp.dot(a, w, preferred_element_type=jnp.float32)

    xi = jnp.maximum(dot(x, fcW_ref[...]) + fcb_ref[...], 0.0)
    z = jnp.maximum(dot(dot(A, xi), W0_ref[...]) + b0_ref[...], 0.0)
    z = jnp.maximum(dot(dot(A, z), W1_ref[...]) + b1_ref[...], 0.0)

    h = jnp.where(t == 0, 0.0, h_s[b])
    r = jax.nn.sigmoid(dot(z, Wr_ref[...]) + dot(h, Ur_ref[...]) + br_ref[...])
    u = jax.nn.sigmoid(dot(z, Wu_ref[...]) + dot(h, Uu_ref[...]) + bu_ref[...])
    c = jnp.tanh(dot(z, Wc_ref[...]) + dot(r * h, Uc_ref[...]) + bc_ref[...])
    hn = u * h + (1.0 - u) * c

    h_s[b] = hn
    out_ref[0, 0] = hn


def kernel(x, e, fc_W, fc_b, W0, b0, W1, b1, Wr, Ur, br, Wu, Uu, bu, Wc, Uc, bc):
    B, T, N, in_ft = x.shape
    out_ft = Ur.shape[0]
    A = e[..., 0]

    row = lambda v: v.reshape(1, -1)
    wspec = lambda s: pl.BlockSpec(s, lambda t, b: (0, 0))

    grid = (T, B)
    out = pl.pallas_call(
        _step,
        grid=grid,
        in_specs=[
            pl.BlockSpec((1, 1, N, in_ft), lambda t, b: (b, t, 0, 0)),
            pl.BlockSpec((1, 1, N, N), lambda t, b: (b, t, 0, 0)),
            wspec(fc_W.shape), wspec((1, fc_b.shape[0])),
            wspec(W0.shape), wspec((1, b0.shape[0])),
            wspec(W1.shape), wspec((1, b1.shape[0])),
            wspec(Wr.shape), wspec(Ur.shape), wspec((1, br.shape[0])),
            wspec(Wu.shape), wspec(Uu.shape), wspec((1, bu.shape[0])),
            wspec(Wc.shape), wspec(Uc.shape), wspec((1, bc.shape[0])),
        ],
        out_specs=pl.BlockSpec((1, 1, N, out_ft), lambda t, b: (b, t, 0, 0)),
        out_shape=jax.ShapeDtypeStruct((B, T, N, out_ft), jnp.float32),
        scratch_shapes=[pltpu.VMEM((B, N, out_ft), jnp.float32)],
    )(x, A, fc_W, row(fc_b), W0, row(b0), W1, row(b1),
      Wr, Ur, row(br), Wu, Uu, row(bu), Wc, Uc, row(bc))
    return out


# speedup vs baseline: 1.1925x; 1.0019x over previous
import jax
import jax.numpy as jnp
from jax.experimental import pallas as pl
from jax.experimental.pallas import tpu as pltpu


def _step(x_ref, e_ref, fcW_ref, fcb_ref, W0_ref, b0_ref, W1_ref, b1_ref,
          Wg_ref, bg_ref, Uru_ref, Uc_ref, out_ref, h_s):
    t = pl.program_id(0)
    b = pl.program_id(1)

    A = e_ref[0, 0]
    x = x_ref[0, 0]
    F = Uc_ref.shape[0]

    dot = lambda a, w: jnp.dot(a, w, preferred_element_type=jnp.float32)

    xi = jnp.maximum(dot(x, fcW_ref[...]) + fcb_ref[...], 0.0)
    z = jnp.maximum(dot(dot(A, xi), W0_ref[...]) + b0_ref[...], 0.0)
    z = jnp.maximum(dot(dot(A, z), W1_ref[...]) + b1_ref[...], 0.0)

    h = jnp.where(t == 0, 0.0, h_s[b])
    g = dot(z, Wg_ref[...]) + bg_ref[...]
    g_ru = g[:, : 2 * F] + dot(h, Uru_ref[...])
    ru = jax.nn.sigmoid(g_ru)
    r = ru[:, :F]
    u = ru[:, F:]
    c = jnp.tanh(g[:, 2 * F:] + dot(r * h, Uc_ref[...]))
    hn = u * h + (1.0 - u) * c

    h_s[b] = hn
    out_ref[0, 0] = hn


def kernel(x, e, fc_W, fc_b, W0, b0, W1, b1, Wr, Ur, br, Wu, Uu, bu, Wc, Uc, bc):
    B, T, N, in_ft = x.shape
    out_ft = Ur.shape[0]
    A = e[..., 0]

    Wg = jnp.concatenate([Wr, Wu, Wc], axis=1)
    bg = jnp.concatenate([br, bu, bc]).reshape(1, -1)
    Uru = jnp.concatenate([Ur, Uu], axis=1)

    row = lambda v: v.reshape(1, -1)
    wspec = lambda s: pl.BlockSpec(s, lambda t, b: (0, 0))

    grid = (T, B)
    out = pl.pallas_call(
        _step,
        grid=grid,
        in_specs=[
            pl.BlockSpec((1, 1, N, in_ft), lambda t, b: (b, t, 0, 0)),
            pl.BlockSpec((1, 1, N, N), lambda t, b: (b, t, 0, 0)),
            wspec(fc_W.shape), wspec((1, fc_b.shape[0])),
            wspec(W0.shape), wspec((1, b0.shape[0])),
            wspec(W1.shape), wspec((1, b1.shape[0])),
            wspec(Wg.shape), wspec(bg.shape), wspec(Uru.shape), wspec(Uc.shape),
        ],
        out_specs=pl.BlockSpec((1, 1, N, out_ft), lambda t, b: (b, t, 0, 0)),
        out_shape=jax.ShapeDtypeStruct((B, T, N, out_ft), jnp.float32),
        scratch_shapes=[pltpu.VMEM((B, N, out_ft), jnp.float32)],
    )(x, A, fc_W, row(fc_b), W0, row(b0), W1, row(b1), Wg, bg, Uru, Uc)
    return out
